# Initial kernel scaffold; baseline (speedup 1.0000x reference)
import jax
import jax.numpy as jnp
from jax.experimental import pallas as pl


def kernel(x, a, a_index):
    raise NotImplementedError("write your pallas kernel here")


# speedup vs baseline: 3.5269x; 3.5269x over previous
import functools

import jax
import jax.numpy as jnp
from jax.experimental import pallas as pl
from jax.experimental.pallas import tpu as pltpu

_ROWS = 16384
_FEATS = 128
_BLOCK_ROWS = 2048


def _body(x_ref, p_ref, a_ref, o_ref):
    z = jax.lax.dot(
        x_ref[...], p_ref[...], precision=jax.lax.Precision.HIGHEST,
        preferred_element_type=jnp.float32,
    )
    z = z - a_ref[0:1, :]
    o_ref[...] = jax.nn.sigmoid(z)


@jax.jit
def kernel(x, a, a_index):
    n, d = x.shape
    p = (a_index[None, :] == jax.lax.iota(jnp.int32, d)[:, None]).astype(x.dtype)
    a_b = jnp.broadcast_to(a, (8, d))
    block = min(_BLOCK_ROWS, n)
    grid = (n // block,)
    return pl.pallas_call(
        _body,
        grid=grid,
        in_specs=[
            pl.BlockSpec((block, d), lambda i: (i, 0)),
            pl.BlockSpec((d, d), lambda i: (0, 0)),
            pl.BlockSpec((8, d), lambda i: (0, 0)),
        ],
        out_specs=pl.BlockSpec((block, d), lambda i: (i, 0)),
        out_shape=jax.ShapeDtypeStruct((n, d), x.dtype),
    )(x, p, a_b)


# speedup vs baseline: 4.2621x; 1.2085x over previous
import functools

import jax
import jax.numpy as jnp
from jax.experimental import pallas as pl
from jax.experimental.pallas import tpu as pltpu

_ROWS = 16384
_FEATS = 128
_BLOCK_ROWS = 2048


def _body(x_ref, i_ref, a_ref, o_ref):
    x = x_ref[...]
    idx = jnp.broadcast_to(i_ref[0:1, :], x.shape)
    z = jnp.take_along_axis(x, idx, axis=1)
    z = z - a_ref[0:1, :]
    o_ref[...] = jax.nn.sigmoid(z)


@jax.jit
def kernel(x, a, a_index):
    n, d = x.shape
    idx_b = jnp.broadcast_to(a_index[None, :], (8, d))
    a_b = jnp.broadcast_to(a, (8, d))
    block = min(_BLOCK_ROWS, n)
    grid = (n // block,)
    return pl.pallas_call(
        _body,
        grid=grid,
        in_specs=[
            pl.BlockSpec((block, d), lambda i: (i, 0)),
            pl.BlockSpec((8, d), lambda i: (0, 0)),
            pl.BlockSpec((8, d), lambda i: (0, 0)),
        ],
        out_specs=pl.BlockSpec((block, d), lambda i: (i, 0)),
        out_shape=jax.ShapeDtypeStruct((n, d), x.dtype),
    )(x, idx_b, a_b)
